# Initial kernel scaffold; baseline (speedup 1.0000x reference)
import jax
import jax.numpy as jnp
from jax.experimental import pallas as pl


def kernel(x, type_emb, index):
    raise NotImplementedError("write your pallas kernel here")


# speedup vs baseline: 4.0459x; 4.0459x over previous
import jax
import jax.numpy as jnp
from jax.experimental import pallas as pl
from jax.experimental.pallas import tpu as pltpu


def _body(idx_ref, x_ref, emb_ref, o_ref):
    i = idx_ref[0]
    row = emb_ref[pl.ds(i, 1), :]
    o_ref[...] = x_ref[...] + row


def kernel(x, type_emb, index):
    B, S, D = x.shape
    N = B * S
    xf = x.reshape(N, D)
    idx = jnp.asarray(index, jnp.int32).reshape(1)

    BM = 1024
    grid = (N // BM,)

    out = pl.pallas_call(
        _body,
        grid_spec=pltpu.PrefetchScalarGridSpec(
            num_scalar_prefetch=1,
            grid=grid,
            in_specs=[
                pl.BlockSpec((BM, D), lambda i, s: (i, 0)),
                pl.BlockSpec((2, D), lambda i, s: (0, 0)),
            ],
            out_specs=pl.BlockSpec((BM, D), lambda i, s: (i, 0)),
        ),
        out_shape=jax.ShapeDtypeStruct((N, D), x.dtype),
    )(idx, xf, type_emb)
    return out.reshape(B, S, D)


# speedup vs baseline: 4.1935x; 1.0365x over previous
import jax
import jax.numpy as jnp
from jax.experimental import pallas as pl
from jax.experimental.pallas import tpu as pltpu


def _body(idx_ref, x_ref, emb_ref, o_ref):
    i = idx_ref[0]
    row = emb_ref[pl.ds(i, 1), :]
    o_ref[...] = x_ref[...] + row


def kernel(x, type_emb, index):
    B, S, D = x.shape
    N = B * S
    xf = x.reshape(N, D)
    idx = jnp.asarray(index, jnp.int32).reshape(1)

    BM = 2048
    grid = (N // BM,)

    out = pl.pallas_call(
        _body,
        grid_spec=pltpu.PrefetchScalarGridSpec(
            num_scalar_prefetch=1,
            grid=grid,
            in_specs=[
                pl.BlockSpec((BM, D), lambda i, s: (i, 0)),
                pl.BlockSpec((2, D), lambda i, s: (0, 0)),
            ],
            out_specs=pl.BlockSpec((BM, D), lambda i, s: (i, 0)),
        ),
        out_shape=jax.ShapeDtypeStruct((N, D), x.dtype),
    )(idx, xf, type_emb)
    return out.reshape(B, S, D)
